# baseline (speedup 1.0000x reference)
import functools

import jax
import jax.numpy as jnp
from jax import lax
from jax.experimental import pallas as pl
from jax.experimental.pallas import tpu as pltpu
from jax.experimental.pallas import tpu_sc as plsc

_LOG2 = 0.6931471805599453

_NC = 2
_NS = 16
_LANES = 16

_CHUNK = 64
_CHUNKS_PER_TILE = 160
_GRP = 40

_ACC_ROWS = 10240

_EPAD = _NC * _NS * _CHUNKS_PER_TILE * _CHUNK
_NPAD = 10240
_ZBLK = 2048


def _z_body(e_ref, off_ref, wid_ref, w2_ref, b2_ref, r_ref, aw_ref, o_ref,
            *, ecount, wblocks):
    pid = pl.program_id(0)

    @pl.when(pid < wblocks)
    def _filter_rows():
        d = (e_ref[:, :] - off_ref[:, :]) / wid_ref[:, :]
        es = jnp.exp(-0.5 * d * d)
        w = jnp.dot(es, w2_ref[:, :],
                    preferred_element_type=jnp.float32) + b2_ref[:, :]
        gidx = pid * _ZBLK + lax.broadcasted_iota(jnp.int32, (_ZBLK, 1), 0)
        o_ref[:, :] = jnp.where(gidx < ecount, w, 0.0)

    @pl.when(pid >= wblocks)
    def _node_rows():
        o_ref[:, :] = jnp.dot(r_ref[:, :], aw_ref[:, :],
                              preferred_element_type=jnp.float32)


def _out_mlp_body(p0_ref, p1_ref, d1_ref, b1_ref, d2_ref, b2_ref, o_ref):
    h = p0_ref[:, :] + p1_ref[:, :]
    t = jnp.dot(h, d1_ref[:, :], preferred_element_type=jnp.float32) + b1_ref[:, :]
    m = jnp.maximum(t, 0.0)
    sp = m + jnp.log(jnp.exp(t - m) + jnp.exp(-m)) - _LOG2
    o_ref[:, :] = jnp.dot(sp, d2_ref[:, :],
                          preferred_element_type=jnp.float32) + b2_ref[:, :]


def _sc_body(z_hbm, dst_hbm, gidx_hbm, out_hbm,
             gidx_v, dst_v, z0, z1, acc, semg, sems):
    c = lax.axis_index("c")
    s = lax.axis_index("s")
    wid = c * _NS + s
    R = _CHUNKS_PER_TILE
    tile_row = wid * R
    dr = _ACC_ROWS // _NS
    full = dr // 128

    def _zero_row(i, _):
        for k in range(8):
            z0[i, pl.ds(k * _LANES, _LANES)] = jnp.zeros((_LANES,), jnp.float32)
        return 0
    lax.fori_loop(0, 128, _zero_row, 0)
    for t in range(full):
        pltpu.sync_copy(z0, acc.at[pl.ds(s * dr + t * 128, 128)])
    plsc.subcore_barrier()

    trow = pl.multiple_of(tile_row, 8)
    pltpu.sync_copy(gidx_hbm.at[pl.ds(trow, _GRP)], gidx_v)
    pltpu.sync_copy(dst_hbm.at[pl.ds(trow, _GRP)], dst_v)

    def _gather(i, buf):
        return pltpu.async_copy(z_hbm.at[gidx_v.at[lax.rem(i, _GRP)]], buf, semg)

    def _mul(zb):
        def _mul_row(i, _):
            for k in range(8):
                sl = pl.ds(k * _LANES, _LANES)
                zb[i, sl] = zb[i, sl] * zb[i + _CHUNK, sl]
            return 0
        lax.fori_loop(0, _CHUNK, _mul_row, 0)

    _gather(0, z0)

    def _phase(i, zb, zo):
        boundary = (lax.rem(i, _GRP) == 0) & (i > 0)

        @pl.when(boundary)
        def _restage():
            pltpu.make_async_copy(zo.at[pl.ds(0, _CHUNK)],
                                  acc.at[dst_v.at[0]], sems).wait()
            base = pl.multiple_of(tile_row + i, 8)
            pltpu.sync_copy(gidx_hbm.at[pl.ds(base, _GRP)], gidx_v)
            pltpu.sync_copy(dst_hbm.at[pl.ds(base, _GRP)], dst_v)
            _gather(i, zb)

        pltpu.make_async_copy(z_hbm.at[gidx_v.at[lax.rem(i, _GRP)]],
                              zb, semg).wait()

        @pl.when((~boundary) & (i >= 1))
        def _drain_scatter():
            pltpu.make_async_copy(zo.at[pl.ds(0, _CHUNK)],
                                  acc.at[dst_v.at[0]], sems).wait()

        @pl.when((i + 1 < R) & (lax.rem(i + 1, _GRP) != 0))
        def _prefetch():
            _gather(i + 1, zo)

        _mul(zb)
        pltpu.async_copy(zb.at[pl.ds(0, _CHUNK)],
                         acc.at[dst_v.at[lax.rem(i, _GRP)]], sems, add=True)

    def _iter(j, _):
        _phase(2 * j, z0, z1)
        _phase(2 * j + 1, z1, z0)
        return 0

    lax.fori_loop(0, R // 2, _iter, 0)
    pltpu.make_async_copy(z1.at[pl.ds(0, _CHUNK)],
                          acc.at[dst_v.at[0]], sems).wait()

    plsc.subcore_barrier()
    out_base = c * _ACC_ROWS + s * dr
    for t in range(full):
        pltpu.sync_copy(acc.at[pl.ds(s * dr + t * 128, 128)], z0)
        pltpu.sync_copy(z0, out_hbm.at[pl.ds(out_base + t * 128, 128)])


def kernel(r, e, a, offsets, widths, df1_W, df1_b, df2_W, df2_b, atom_W,
           d1_W, d1_b, d2_W, d2_b):
    n, nab = r.shape
    nf = atom_W.shape[1]
    ng = offsets.shape[0]
    e_count = e.shape[0]
    wblocks = _EPAD // _ZBLK
    nblocks = _NPAD // _ZBLK

    gpad = 128
    off_p = jnp.concatenate([offsets, jnp.zeros((gpad - ng,), jnp.float32)])[None, :]
    wid_p = jnp.concatenate([widths, jnp.ones((gpad - ng,), jnp.float32)])[None, :]
    w2_p = jnp.concatenate(
        [df2_W, jnp.zeros((gpad - ng, nf), jnp.float32)], axis=0)
    e_p = jnp.concatenate(
        [e[:, 0], jnp.zeros((_EPAD - e_count,), jnp.float32)])[:, None]
    r_p = jnp.concatenate([r, jnp.zeros((_NPAD - n, nab), jnp.float32)])

    z = pl.pallas_call(
        functools.partial(_z_body, ecount=e_count, wblocks=wblocks),
        grid=(wblocks + nblocks,),
        in_specs=[
            pl.BlockSpec((_ZBLK, 1), lambda i: (jnp.minimum(i, wblocks - 1), 0)),
            pl.BlockSpec((1, gpad), lambda i: (0, 0)),
            pl.BlockSpec((1, gpad), lambda i: (0, 0)),
            pl.BlockSpec((gpad, nf), lambda i: (0, 0)),
            pl.BlockSpec((1, nf), lambda i: (0, 0)),
            pl.BlockSpec((_ZBLK, nab),
                         lambda i: (jnp.maximum(i - wblocks, 0), 0)),
            pl.BlockSpec((nab, nf), lambda i: (0, 0)),
        ],
        out_specs=pl.BlockSpec((_ZBLK, nf), lambda i: (i, 0)),
        out_shape=jax.ShapeDtypeStruct((_EPAD + _NPAD, nf), jnp.float32),
    )(e_p, off_p, wid_p, w2_p, df2_b[None, :], r_p, atom_W)

    pad = _EPAD - e_count
    dst = jnp.concatenate(
        [a[:, 0], jnp.zeros((pad,), jnp.int32)]).reshape(-1, _CHUNK)
    src = jnp.concatenate(
        [a[:, 1], jnp.zeros((pad,), jnp.int32)]).reshape(-1, _CHUNK)
    gidx = jnp.concatenate(
        [src + _EPAD, jnp.arange(_EPAD, dtype=jnp.int32).reshape(-1, _CHUNK)],
        axis=1)

    sc_fn = pl.kernel(
        _sc_body,
        out_type=jax.ShapeDtypeStruct((_NC * _ACC_ROWS, nf), jnp.float32),
        mesh=plsc.VectorSubcoreMesh(core_axis_name="c", subcore_axis_name="s"),
        scratch_types=[
            pltpu.VMEM((_GRP, 2 * _CHUNK), jnp.int32),
            pltpu.VMEM((_GRP, _CHUNK), jnp.int32),
            pltpu.VMEM((2 * _CHUNK, nf), jnp.float32),
            pltpu.VMEM((2 * _CHUNK, nf), jnp.float32),
            pltpu.VMEM_SHARED((_ACC_ROWS, nf), jnp.float32),
            pltpu.SemaphoreType.DMA,
            pltpu.SemaphoreType.DMA,
        ],
    )
    partials = sc_fn(z, dst, gidx)

    p0 = partials[0:n]
    p1 = partials[_ACC_ROWS:_ACC_ROWS + n]
    rblk = 1000
    out = pl.pallas_call(
        _out_mlp_body,
        grid=(n // rblk,),
        in_specs=[
            pl.BlockSpec((rblk, nf), lambda i: (i, 0)),
            pl.BlockSpec((rblk, nf), lambda i: (i, 0)),
            pl.BlockSpec((nf, nab), lambda i: (0, 0)),
            pl.BlockSpec((1, nab), lambda i: (0, 0)),
            pl.BlockSpec((nab, nab), lambda i: (0, 0)),
            pl.BlockSpec((1, nab), lambda i: (0, 0)),
        ],
        out_specs=pl.BlockSpec((rblk, nab), lambda i: (i, 0)),
        out_shape=jax.ShapeDtypeStruct((n, nab), jnp.float32),
    )(p0, p1, d1_W, d1_b[None, :], d2_W, d2_b[None, :])
    return out
